# baseline (speedup 1.0000x reference)
import functools

import jax
import jax.numpy as jnp
from jax import lax
from jax.experimental import pallas as pl
from jax.experimental.pallas import tpu as pltpu
from jax.experimental.pallas import tpu_sc as plsc

SEQ = 2048
BATCH = 64
DIM = 128
LANES = 16
VPR = DIM // LANES

NC = 2
NS = 16
NW = NC * NS

N = SEQ * BATCH
ROWS_W = N // NW
POS_W = SEQ // NW
CHUNK = 512
POS_CHUNK = CHUNK // BATCH
NCHUNK = ROWS_W // CHUNK


def _make_pe():
    pos = jnp.arange(SEQ, dtype=jnp.float32)[:, None]
    div_term = 1.0 / jnp.power(
        10000.0, jnp.arange(0, DIM * 2, 2, dtype=jnp.float32) / DIM
    )
    pe = pos * div_term[None, :]
    pe = pe.at[:, 0::2].set(jnp.sin(pe[:, 0::2]))
    pe = pe.at[:, 1::2].set(jnp.cos(pe[:, 1::2]))
    return pe


@functools.partial(
    pl.kernel,
    mesh=plsc.VectorSubcoreMesh(core_axis_name="c", subcore_axis_name="s"),
    out_type=jax.ShapeDtypeStruct((N, DIM), jnp.float32),
    scratch_types=[
        pltpu.VMEM((CHUNK,), jnp.int32),
        pltpu.VMEM((CHUNK, DIM), jnp.float32),
        pltpu.VMEM((POS_W, DIM), jnp.float32),
        pltpu.SemaphoreType.DMA,
    ],
)
def _emb_lookup(idx_hbm, table_hbm, pe_hbm, out_hbm, idx_v, rows_v, pe_v, sem):
    wid = lax.axis_index("s") * NC + lax.axis_index("c")
    base = wid * ROWS_W
    pltpu.sync_copy(pe_hbm.at[pl.ds(wid * POS_W, POS_W)], pe_v)

    def chunk_body(c, _):
        cbase = base + c * CHUNK
        pltpu.sync_copy(idx_hbm.at[pl.ds(cbase, CHUNK)], idx_v)
        pltpu.async_copy(table_hbm.at[idx_v], rows_v, sem).wait()

        def pos_body(p, _):
            lp = c * POS_CHUNK + p

            def row_body(r, _):
                row = p * BATCH + r
                for v in range(VPR):
                    plsc.addupdate(
                        rows_v.at[row, pl.ds(v * LANES, LANES)],
                        pe_v[lp, pl.ds(v * LANES, LANES)],
                    )
                return 0

            lax.fori_loop(0, BATCH, row_body, 0)
            return 0

        lax.fori_loop(0, POS_CHUNK, pos_body, 0)
        pltpu.sync_copy(rows_v, out_hbm.at[pl.ds(cbase, CHUNK)])
        return 0

    lax.fori_loop(0, NCHUNK, chunk_body, 0)


def kernel(input, table):
    idx = input[..., 0].reshape(N)
    pe = _make_pe()
    out = _emb_lookup(idx, table, pe)
    return out.reshape(SEQ, BATCH, DIM)


# speedup vs baseline: 2.5453x; 2.5453x over previous
import functools

import jax
import jax.numpy as jnp
from jax import lax
from jax.experimental import pallas as pl
from jax.experimental.pallas import tpu as pltpu
from jax.experimental.pallas import tpu_sc as plsc

SEQ = 2048
BATCH = 64
DIM = 128
LANES = 16
VPR = DIM // LANES

NC = 2
NS = 16
NW = NC * NS

N = SEQ * BATCH
ROWS_W = N // NW
POS_W = SEQ // NW
CHUNK = 256
POS_CHUNK = CHUNK // BATCH
NCHUNK = ROWS_W // CHUNK
UNROLL = 4


def _make_pe():
    pos = jnp.arange(SEQ, dtype=jnp.float32)[:, None]
    div_term = 1.0 / jnp.power(
        10000.0, jnp.arange(0, DIM * 2, 2, dtype=jnp.float32) / DIM
    )
    pe = pos * div_term[None, :]
    pe = pe.at[:, 0::2].set(jnp.sin(pe[:, 0::2]))
    pe = pe.at[:, 1::2].set(jnp.cos(pe[:, 1::2]))
    return pe


@functools.partial(
    pl.kernel,
    mesh=plsc.VectorSubcoreMesh(core_axis_name="c", subcore_axis_name="s"),
    out_type=jax.ShapeDtypeStruct((N, DIM), jnp.float32),
    scratch_types=[
        pltpu.VMEM((ROWS_W,), jnp.int32),
        pltpu.VMEM((CHUNK, DIM), jnp.float32),
        pltpu.VMEM((CHUNK, DIM), jnp.float32),
        pltpu.VMEM((POS_W, DIM), jnp.float32),
        pltpu.SemaphoreType.DMA,
        pltpu.SemaphoreType.DMA,
    ],
)
def _emb_lookup(idx_hbm, table_hbm, pe_hbm, out_hbm,
                idx_v, rows0, rows1, pe_v, sem0, sem1):
    wid = lax.axis_index("s") * NC + lax.axis_index("c")
    base = wid * ROWS_W
    pltpu.sync_copy(idx_hbm.at[pl.ds(base, ROWS_W)], idx_v)
    pltpu.sync_copy(pe_hbm.at[pl.ds(wid * POS_W, POS_W)], pe_v)

    def start_gather(c, rows, sem):
        return pltpu.async_copy(
            table_hbm.at[idx_v.at[pl.ds(c * CHUNK, CHUNK)]], rows, sem
        )

    def add_pe_and_store(c, rows):
        def pos_body(p, _):
            lp = c * POS_CHUNK + p
            pe_vecs = tuple(
                pe_v[lp, pl.ds(v * LANES, LANES)] for v in range(VPR)
            )

            def row_body(r, _):
                row = p * BATCH + r * UNROLL
                for u in range(UNROLL):
                    for v in range(VPR):
                        plsc.addupdate(
                            rows.at[row + u, pl.ds(v * LANES, LANES)],
                            pe_vecs[v],
                        )
                return 0

            lax.fori_loop(0, BATCH // UNROLL, row_body, 0)
            return 0

        lax.fori_loop(0, POS_CHUNK, pos_body, 0)
        pltpu.sync_copy(rows, out_hbm.at[pl.ds(base + c * CHUNK, CHUNK)])

    g0 = start_gather(0, rows0, sem0)
    g1 = start_gather(1, rows1, sem1)

    def chunk_pair(t, _):
        c0 = 2 * t
        g0.wait()
        add_pe_and_store(c0, rows0)

        @pl.when(c0 + 2 < NCHUNK)
        def _():
            start_gather(c0 + 2, rows0, sem0)

        g1.wait()
        add_pe_and_store(c0 + 1, rows1)

        @pl.when(c0 + 3 < NCHUNK)
        def _():
            start_gather(c0 + 3, rows1, sem1)

        return 0

    lax.fori_loop(0, NCHUNK // 2, chunk_pair, 0)


def kernel(input, table):
    idx = input[..., 0].reshape(N)
    pe = _make_pe()
    out = _emb_lookup(idx, table, pe)
    return out.reshape(SEQ, BATCH, DIM)


# speedup vs baseline: 2.7238x; 1.0701x over previous
import functools

import jax
import jax.numpy as jnp
import numpy as np
from jax import lax
from jax.experimental import pallas as pl
from jax.experimental.pallas import tpu as pltpu
from jax.experimental.pallas import tpu_sc as plsc

SEQ = 2048
BATCH = 64
DIM = 128
LANES = 16
VPR = DIM // LANES

NC = 2
NS = 16
NW = NC * NS

N = SEQ * BATCH
ROWS_W = N // NW
POS_W = SEQ // NW
CHUNK = 256
POS_CHUNK = CHUNK // BATCH
NCHUNK = ROWS_W // CHUNK
UNROLL = 4


def _make_pe():
    pos = np.arange(SEQ, dtype=np.float32)[:, None]
    div_term = 1.0 / np.power(
        10000.0, np.arange(0, DIM * 2, 2, dtype=np.float32) / DIM
    )
    pe = pos * div_term[None, :]
    pe[:, 0::2] = np.sin(pe[:, 0::2])
    pe[:, 1::2] = np.cos(pe[:, 1::2])
    return jnp.asarray(pe)


@functools.partial(
    pl.kernel,
    mesh=plsc.VectorSubcoreMesh(core_axis_name="c", subcore_axis_name="s"),
    out_type=jax.ShapeDtypeStruct((N, DIM), jnp.float32),
    scratch_types=[
        pltpu.VMEM((ROWS_W,), jnp.int32),
        pltpu.VMEM((CHUNK, DIM), jnp.float32),
        pltpu.VMEM((CHUNK, DIM), jnp.float32),
        pltpu.VMEM((POS_W, DIM), jnp.float32),
        pltpu.SemaphoreType.DMA,
        pltpu.SemaphoreType.DMA,
    ],
)
def _emb_lookup(idx_hbm, table_hbm, pe_hbm, out_hbm,
                idx_v, rows0, rows1, pe_v, sem0, sem1):
    wid = lax.axis_index("s") * NC + lax.axis_index("c")
    base = wid * ROWS_W
    pltpu.sync_copy(idx_hbm.at[pl.ds(base, ROWS_W)], idx_v)
    pltpu.sync_copy(pe_hbm.at[pl.ds(wid * POS_W, POS_W)], pe_v)

    def start_gather(c, rows, sem):
        return pltpu.async_copy(
            table_hbm.at[idx_v.at[pl.ds(c * CHUNK, CHUNK)]], rows, sem
        )

    def add_pe_and_store(c, rows):
        def pos_body(p, _):
            lp = c * POS_CHUNK + p
            pe_vecs = tuple(
                pe_v[lp, pl.ds(v * LANES, LANES)] for v in range(VPR)
            )

            def row_body(r, _):
                row = p * BATCH + r * UNROLL
                for u in range(UNROLL):
                    for v in range(VPR):
                        plsc.addupdate(
                            rows.at[row + u, pl.ds(v * LANES, LANES)],
                            pe_vecs[v],
                        )
                return 0

            lax.fori_loop(0, BATCH // UNROLL, row_body, 0)
            return 0

        lax.fori_loop(0, POS_CHUNK, pos_body, 0)
        pltpu.sync_copy(rows, out_hbm.at[pl.ds(base + c * CHUNK, CHUNK)])

    g0 = start_gather(0, rows0, sem0)
    g1 = start_gather(1, rows1, sem1)

    def chunk_pair(t, _):
        c0 = 2 * t
        g0.wait()
        add_pe_and_store(c0, rows0)

        @pl.when(c0 + 2 < NCHUNK)
        def _():
            start_gather(c0 + 2, rows0, sem0)

        g1.wait()
        add_pe_and_store(c0 + 1, rows1)

        @pl.when(c0 + 3 < NCHUNK)
        def _():
            start_gather(c0 + 3, rows1, sem1)

        return 0

    lax.fori_loop(0, NCHUNK // 2, chunk_pair, 0)


def kernel(input, table):
    idx = input[..., 0].reshape(N)
    pe = _make_pe()
    out = _emb_lookup(idx, table, pe)
    return out.reshape(SEQ, BATCH, DIM)


# speedup vs baseline: 2.7800x; 1.0206x over previous
import functools

import jax
import jax.numpy as jnp
import numpy as np
from jax import lax
from jax.experimental import pallas as pl
from jax.experimental.pallas import tpu as pltpu
from jax.experimental.pallas import tpu_sc as plsc

SEQ = 2048
BATCH = 64
DIM = 128
LANES = 16
VPR = DIM // LANES

NC = 2
NS = 16
NW = NC * NS

N = SEQ * BATCH
ROWS_W = N // NW
POS_W = SEQ // NW
CHUNK = 128
POS_CHUNK = CHUNK // BATCH
NCHUNK = ROWS_W // CHUNK
NBUF = 4
UNROLL = 4


def _make_pe():
    pos = np.arange(SEQ, dtype=np.float32)[:, None]
    div_term = 1.0 / np.power(
        10000.0, np.arange(0, DIM * 2, 2, dtype=np.float32) / DIM
    )
    pe = pos * div_term[None, :]
    pe[:, 0::2] = np.sin(pe[:, 0::2])
    pe[:, 1::2] = np.cos(pe[:, 1::2])
    return jnp.asarray(pe)


@functools.partial(
    pl.kernel,
    mesh=plsc.VectorSubcoreMesh(core_axis_name="c", subcore_axis_name="s"),
    out_type=jax.ShapeDtypeStruct((N, DIM), jnp.float32),
    scratch_types=(
        [pltpu.VMEM((ROWS_W,), jnp.int32)]
        + [pltpu.VMEM((CHUNK, DIM), jnp.float32) for _ in range(NBUF)]
        + [pltpu.VMEM((POS_W, DIM), jnp.float32)]
        + [pltpu.SemaphoreType.DMA for _ in range(2 * NBUF)]
    ),
)
def _emb_lookup(idx_hbm, table_hbm, pe_hbm, out_hbm, idx_v, *rest):
    rows = rest[:NBUF]
    pe_v = rest[NBUF]
    gsem = rest[NBUF + 1:NBUF + 1 + NBUF]
    wsem = rest[NBUF + 1 + NBUF:]

    wid = lax.axis_index("s") * NC + lax.axis_index("c")
    base = wid * ROWS_W
    pltpu.sync_copy(idx_hbm.at[pl.ds(base, ROWS_W)], idx_v)
    pltpu.sync_copy(pe_hbm.at[pl.ds(wid * POS_W, POS_W)], pe_v)

    def start_gather(c, b):
        return pltpu.async_copy(
            table_hbm.at[idx_v.at[pl.ds(c * CHUNK, CHUNK)]], rows[b], gsem[b]
        )

    def start_wb(c, b):
        return pltpu.async_copy(
            rows[b], out_hbm.at[pl.ds(base + c * CHUNK, CHUNK)], wsem[b]
        )

    def add_pe(c, b):
        def pos_body(p, _):
            lp = c * POS_CHUNK + p
            pe_vecs = tuple(
                pe_v[lp, pl.ds(v * LANES, LANES)] for v in range(VPR)
            )

            def row_body(r, _):
                row = p * BATCH + r * UNROLL
                for u in range(UNROLL):
                    for v in range(VPR):
                        plsc.addupdate(
                            rows[b].at[row + u, pl.ds(v * LANES, LANES)],
                            pe_vecs[v],
                        )
                return 0

            lax.fori_loop(0, BATCH // UNROLL, row_body, 0)
            return 0

        lax.fori_loop(0, POS_CHUNK, pos_body, 0)

    for b in range(NBUF - 1):
        start_gather(b, b)

    def ring_body(t, _):
        for b in range(NBUF):
            c = t * NBUF + b
            pltpu.make_async_copy(
                table_hbm.at[idx_v.at[pl.ds(0, CHUNK)]], rows[b], gsem[b]
            ).wait()
            add_pe(c, b)
            start_wb(c, b)
            nb = (b + NBUF - 1) % NBUF
            nc = c + NBUF - 1

            @pl.when(nc < NCHUNK)
            def _():
                @pl.when(nc >= NBUF)
                def _():
                    pltpu.make_async_copy(
                        rows[nb], out_hbm.at[pl.ds(base, CHUNK)], wsem[nb]
                    ).wait()

                start_gather(nc, nb)

        return 0

    lax.fori_loop(0, NCHUNK // NBUF, ring_body, 0)

    for b in range(NBUF):
        pltpu.make_async_copy(
            rows[b], out_hbm.at[pl.ds(base, CHUNK)], wsem[b]
        ).wait()


def kernel(input, table):
    idx = input[..., 0].reshape(N)
    pe = _make_pe()
    out = _emb_lookup(idx, table, pe)
    return out.reshape(SEQ, BATCH, DIM)


# speedup vs baseline: 2.8537x; 1.0265x over previous
import functools

import jax
import jax.numpy as jnp
import numpy as np
from jax import lax
from jax.experimental import pallas as pl
from jax.experimental.pallas import tpu as pltpu
from jax.experimental.pallas import tpu_sc as plsc

SEQ = 2048
BATCH = 64
DIM = 128
LANES = 16
VPR = DIM // LANES

NC = 2
NS = 16
NW = NC * NS

N = SEQ * BATCH
ROWS_W = N // NW
POS_W = SEQ // NW
CHUNK = 64
POS_CHUNK = CHUNK // BATCH
NCHUNK = ROWS_W // CHUNK
NBUF = 8
UNROLL = 4


def _make_pe():
    pos = np.arange(SEQ, dtype=np.float32)[:, None]
    div_term = 1.0 / np.power(
        10000.0, np.arange(0, DIM * 2, 2, dtype=np.float32) / DIM
    )
    pe = pos * div_term[None, :]
    pe[:, 0::2] = np.sin(pe[:, 0::2])
    pe[:, 1::2] = np.cos(pe[:, 1::2])
    return jnp.asarray(pe)


@functools.partial(
    pl.kernel,
    mesh=plsc.VectorSubcoreMesh(core_axis_name="c", subcore_axis_name="s"),
    out_type=jax.ShapeDtypeStruct((N, DIM), jnp.float32),
    scratch_types=(
        [pltpu.VMEM((ROWS_W,), jnp.int32)]
        + [pltpu.VMEM((CHUNK, DIM), jnp.float32) for _ in range(NBUF)]
        + [pltpu.VMEM((POS_W, DIM), jnp.float32)]
        + [pltpu.SemaphoreType.DMA for _ in range(2 * NBUF)]
    ),
)
def _emb_lookup(idx_hbm, table_hbm, pe_hbm, out_hbm, idx_v, *rest):
    rows = rest[:NBUF]
    pe_v = rest[NBUF]
    gsem = rest[NBUF + 1:NBUF + 1 + NBUF]
    wsem = rest[NBUF + 1 + NBUF:]

    wid = lax.axis_index("s") * NC + lax.axis_index("c")
    base = wid * ROWS_W
    pltpu.sync_copy(idx_hbm.at[pl.ds(base, ROWS_W)], idx_v)
    pltpu.sync_copy(pe_hbm.at[pl.ds(wid * POS_W, POS_W)], pe_v)

    def start_gather(c, b):
        return pltpu.async_copy(
            table_hbm.at[idx_v.at[pl.ds(c * CHUNK, CHUNK)]], rows[b], gsem[b]
        )

    def start_wb(c, b):
        return pltpu.async_copy(
            rows[b], out_hbm.at[pl.ds(base + c * CHUNK, CHUNK)], wsem[b]
        )

    def add_pe(c, b):
        def pos_body(p, _):
            lp = c * POS_CHUNK + p
            pe_vecs = tuple(
                pe_v[lp, pl.ds(v * LANES, LANES)] for v in range(VPR)
            )

            def row_body(r, _):
                row = p * BATCH + r * UNROLL
                for u in range(UNROLL):
                    for v in range(VPR):
                        plsc.addupdate(
                            rows[b].at[row + u, pl.ds(v * LANES, LANES)],
                            pe_vecs[v],
                        )
                return 0

            lax.fori_loop(0, BATCH // UNROLL, row_body, 0)
            return 0

        lax.fori_loop(0, POS_CHUNK, pos_body, 0)

    for b in range(NBUF - 1):
        start_gather(b, b)

    def ring_body(t, _):
        for b in range(NBUF):
            c = t * NBUF + b
            pltpu.make_async_copy(
                table_hbm.at[idx_v.at[pl.ds(0, CHUNK)]], rows[b], gsem[b]
            ).wait()
            add_pe(c, b)
            start_wb(c, b)
            nb = (b + NBUF - 1) % NBUF
            nc = c + NBUF - 1

            @pl.when(nc < NCHUNK)
            def _():
                @pl.when(nc >= NBUF)
                def _():
                    pltpu.make_async_copy(
                        rows[nb], out_hbm.at[pl.ds(base, CHUNK)], wsem[nb]
                    ).wait()

                start_gather(nc, nb)

        return 0

    lax.fori_loop(0, NCHUNK // NBUF, ring_body, 0)

    for b in range(NBUF):
        pltpu.make_async_copy(
            rows[b], out_hbm.at[pl.ds(base, CHUNK)], wsem[b]
        ).wait()


def kernel(input, table):
    idx = input[..., 0].reshape(N)
    pe = _make_pe()
    out = _emb_lookup(idx, table, pe)
    return out.reshape(SEQ, BATCH, DIM)
